# Initial kernel scaffold; baseline (speedup 1.0000x reference)
import jax
import jax.numpy as jnp
from jax.experimental import pallas as pl


def kernel(points, W1, b1, W2, b2, Wf1, bf1, Wf2, bf2, gamma, beta):
    raise NotImplementedError("write your pallas kernel here")


# speedup vs baseline: 5.0908x; 5.0908x over previous
import functools

import jax
import jax.numpy as jnp
from jax.experimental import pallas as pl
from jax.experimental.pallas import tpu as pltpu
from jax.experimental.pallas import tpu_sc as plsc


def _gelu(x):
    return x * 0.5 * (1.0 + jax.lax.erf(x * 0.7071067811865476))


def _sc_fps(flat_pts, B, N, S):
    mesh = plsc.VectorSubcoreMesh(core_axis_name="c", subcore_axis_name="s")

    @functools.partial(
        pl.kernel,
        out_type=jax.ShapeDtypeStruct((B * 3 * S,), jnp.float32),
        mesh=mesh,
        compiler_params=pltpu.CompilerParams(needs_layout_passes=False),
        scratch_types=[
            pltpu.VMEM((N,), jnp.float32),
            pltpu.VMEM((N,), jnp.float32),
            pltpu.VMEM((N,), jnp.float32),
            pltpu.VMEM((N,), jnp.float32),
            pltpu.VMEM((S,), jnp.float32),
            pltpu.VMEM((S,), jnp.float32),
            pltpu.VMEM((S,), jnp.float32),
        ],
    )
    def fps_kernel(pts_hbm, out_hbm, px, py, pz, mind, selx, sely, selz):
        cid = jax.lax.axis_index("c")
        sid = jax.lax.axis_index("s")
        wid = sid * 2 + cid

        @pl.when(wid < B)
        def _():
            b = wid
            pltpu.sync_copy(pts_hbm.at[pl.ds((3 * b + 0) * N, N)], px)
            pltpu.sync_copy(pts_hbm.at[pl.ds((3 * b + 1) * N, N)], py)
            pltpu.sync_copy(pts_hbm.at[pl.ds((3 * b + 2) * N, N)], pz)

            inf16 = jnp.full((16,), jnp.inf, jnp.float32)
            ninf = jnp.float32(-jnp.inf)

            def init_body(i, carry):
                mind[pl.ds(i * 16, 16)] = inf16
                return carry

            jax.lax.fori_loop(0, N // 16, init_body, 0)

            lane = jax.lax.iota(jnp.int32, 16)
            mask0 = lane == 0
            first = px[pl.ds(0, 16)]
            sx = jnp.full((16,), jnp.max(jnp.where(mask0, first, ninf)))
            first = py[pl.ds(0, 16)]
            sy = jnp.full((16,), jnp.max(jnp.where(mask0, first, ninf)))
            first = pz[pl.ds(0, 16)]
            sz = jnp.full((16,), jnp.max(jnp.where(mask0, first, ninf)))

            def step(t, carry):
                sx, sy, sz = carry
                tv = jnp.full((16,), t, jnp.int32)
                plsc.store_scatter(selx, [tv], sx, mask=mask0)
                plsc.store_scatter(sely, [tv], sy, mask=mask0)
                plsc.store_scatter(selz, [tv], sz, mask=mask0)

                def inner(i, c2):
                    vmax, vidx, vx, vy, vz = c2
                    sl = pl.ds(i * 16, 16)
                    x = px[sl]
                    y = py[sl]
                    z = pz[sl]
                    dx = x - sx
                    dy = y - sy
                    dz = z - sz
                    d = (dx * dx + dy * dy) + dz * dz
                    m = jnp.minimum(mind[sl], d)
                    mind[sl] = m
                    upd = m > vmax
                    vmax = jnp.where(upd, m, vmax)
                    vidx = jnp.where(upd, i * 16 + lane, vidx)
                    vx = jnp.where(upd, x, vx)
                    vy = jnp.where(upd, y, vy)
                    vz = jnp.where(upd, z, vz)
                    return vmax, vidx, vx, vy, vz

                vmax0 = jnp.full((16,), ninf)
                zidx = jnp.zeros((16,), jnp.int32)
                zf = jnp.zeros((16,), jnp.float32)
                vmax, vidx, vx, vy, vz = jax.lax.fori_loop(
                    0, N // 16, inner, (vmax0, zidx, zf, zf, zf))
                gmax = jnp.max(vmax)
                cand = jnp.where(vmax == gmax, vidx, jnp.int32(N))
                gidx = jnp.min(cand)
                win = vidx == gidx
                return (
                    jnp.full((16,), jnp.max(jnp.where(win, vx, ninf))),
                    jnp.full((16,), jnp.max(jnp.where(win, vy, ninf))),
                    jnp.full((16,), jnp.max(jnp.where(win, vz, ninf))),
                )

            jax.lax.fori_loop(0, S, step, (sx, sy, sz))

            pltpu.sync_copy(selx, out_hbm.at[pl.ds((3 * b + 0) * S, S)])
            pltpu.sync_copy(sely, out_hbm.at[pl.ds((3 * b + 1) * S, S)])
            pltpu.sync_copy(selz, out_hbm.at[pl.ds((3 * b + 2) * S, S)])

    return fps_kernel(flat_pts)


def _tc_feat_sums(ptsT, W1T, b1c, W2T, b2c, B, N, CH):
    nch = N // CH

    def body(p_ref, w1_ref, b1_ref, w2_ref, b2_ref, out_ref):
        i = pl.program_id(1)

        @pl.when(i == 0)
        def _():
            out_ref[...] = jnp.zeros_like(out_ref)

        pts = p_ref[0]
        h = _gelu(jnp.dot(w1_ref[...], pts, preferred_element_type=jnp.float32)
                  + b1_ref[...])
        f = _gelu(jnp.dot(w2_ref[...], h, preferred_element_type=jnp.float32)
                  + b2_ref[...])
        out_ref[...] += jnp.sum(f, axis=1)[None, None, :]

    D = W2T.shape[0]
    return pl.pallas_call(
        body,
        grid=(B, nch),
        in_specs=[
            pl.BlockSpec((1, 3, CH), lambda b, i: (b, 0, i)),
            pl.BlockSpec((64, 3), lambda b, i: (0, 0)),
            pl.BlockSpec((64, 1), lambda b, i: (0, 0)),
            pl.BlockSpec((D, 64), lambda b, i: (0, 0)),
            pl.BlockSpec((D, 1), lambda b, i: (0, 0)),
        ],
        out_specs=pl.BlockSpec((1, 1, D), lambda b, i: (b, 0, 0)),
        out_shape=jax.ShapeDtypeStruct((B, 1, D), jnp.float32),
    )(ptsT, W1T, b1c, W2T, b2c)


def _tc_tail(sel, sums, W1, b1, W2, b2, Wf1a, Wf1b, bf1, Wf2, bf2, gamma, beta,
             B, N, S):
    D = W2.shape[1]

    def body(sel_ref, sums_ref, w1_ref, b1_ref, w2_ref, b2_ref, wa_ref, wb_ref,
             bf1_ref, wf2_ref, bf2_ref, g_ref, be_ref, out_ref):
        sp = sel_ref[0]
        sums_row = sums_ref[0]
        h = _gelu(jnp.dot(sp, w1_ref[...], preferred_element_type=jnp.float32)
                  + b1_ref[...])
        f = _gelu(jnp.dot(h, w2_ref[...], preferred_element_type=jnp.float32)
                  + b2_ref[...])
        mrow = sums_row * (1.0 / N)
        ctx = jnp.dot(mrow, wb_ref[...], preferred_element_type=jnp.float32)
        pre = (jnp.dot(f, wa_ref[...], preferred_element_type=jnp.float32)
               + ctx + bf1_ref[...])
        t = (jnp.dot(_gelu(pre), wf2_ref[...], preferred_element_type=jnp.float32)
             + bf2_ref[...])
        mu = jnp.mean(t, axis=1, keepdims=True)
        c = t - mu
        var = jnp.mean(c * c, axis=1, keepdims=True)
        out_ref[0] = c / jnp.sqrt(var + 1e-5) * g_ref[...] + be_ref[...]

    return pl.pallas_call(
        body,
        grid=(B,),
        in_specs=[
            pl.BlockSpec((1, S, 3), lambda b: (b, 0, 0)),
            pl.BlockSpec((1, 1, D), lambda b: (b, 0, 0)),
            pl.BlockSpec((3, 64), lambda b: (0, 0)),
            pl.BlockSpec((1, 64), lambda b: (0, 0)),
            pl.BlockSpec((64, D), lambda b: (0, 0)),
            pl.BlockSpec((1, D), lambda b: (0, 0)),
            pl.BlockSpec((D, D), lambda b: (0, 0)),
            pl.BlockSpec((D, D), lambda b: (0, 0)),
            pl.BlockSpec((1, D), lambda b: (0, 0)),
            pl.BlockSpec((D, D), lambda b: (0, 0)),
            pl.BlockSpec((1, D), lambda b: (0, 0)),
            pl.BlockSpec((1, D), lambda b: (0, 0)),
            pl.BlockSpec((1, D), lambda b: (0, 0)),
        ],
        out_specs=pl.BlockSpec((1, S, D), lambda b: (b, 0, 0)),
        out_shape=jax.ShapeDtypeStruct((B, S, D), jnp.float32),
    )(sel, sums, W1, b1, W2, b2, Wf1a, Wf1b, bf1, Wf2, bf2, gamma, beta)


def kernel(points, W1, b1, W2, b2, Wf1, bf1, Wf2, bf2, gamma, beta):
    B, N, _ = points.shape
    S = 256
    D = W2.shape[1]

    ptsT = jnp.transpose(points, (0, 2, 1))
    flat_pts = ptsT.reshape(B * 3 * N)

    sel_flat = _sc_fps(flat_pts, B, N, S)
    sums = _tc_feat_sums(
        ptsT, jnp.transpose(W1), b1.reshape(-1, 1),
        jnp.transpose(W2), b2.reshape(-1, 1), B, N, 2048)

    sel = jnp.transpose(sel_flat.reshape(B, 3, S), (0, 2, 1))
    out = _tc_tail(
        sel, sums, W1, b1.reshape(1, -1), W2, b2.reshape(1, -1),
        Wf1[:D], Wf1[D:], bf1.reshape(1, -1), Wf2, bf2.reshape(1, -1),
        gamma.reshape(1, -1), beta.reshape(1, -1), B, N, S)
    return out
